# baseline (speedup 1.0000x reference)
import functools

import jax
import jax.numpy as jnp
from jax import lax
from jax.experimental import pallas as pl
from jax.experimental.pallas import tpu as pltpu
from jax.experimental.pallas import tpu_sc as plsc

B, Q, T = 64, 900, 256
C = 900
K = 300
PADC = 1024
CHUNK = 128
NCH = Q * PADC // CHUNK
CCAP = 384
NCAND = CCAP * CHUNK
SCAP = 512
FINF_BITS = 0x7F800000
NC_SC, NS_SC = 2, 16
NW = NC_SC * NS_SC
NEG = -jnp.inf


def _k1_body(logits_ref, pmap_ref, probp_ref, cm_ref):
    x = logits_ref[0]
    s = jax.nn.sigmoid(x)
    p = lax.dot_general(s, pmap_ref[...], (((1,), (1,)), ((), ())),
                        preferred_element_type=jnp.float32)
    pad = jnp.full((Q, PADC - C), NEG, jnp.float32)
    probp_ref[0] = jnp.concatenate([p, pad], axis=1)
    cms = [jnp.max(p[:, 128 * i:128 * (i + 1)], axis=1, keepdims=True)
           for i in range(7)]
    cms.append(jnp.max(p[:, 896:900], axis=1, keepdims=True))
    cm_ref[0] = jnp.concatenate(cms, axis=1)


def _score_map(pred_logits, positive_map):
    return pl.pallas_call(
        _k1_body,
        grid=(B,),
        in_specs=[
            pl.BlockSpec((1, Q, T), lambda b: (b, 0, 0)),
            pl.BlockSpec((Q, T), lambda b: (0, 0)),
        ],
        out_specs=[
            pl.BlockSpec((1, Q, PADC), lambda b: (b, 0, 0)),
            pl.BlockSpec((1, Q, 8), lambda b: (b, 0, 0)),
        ],
        out_shape=[
            jax.ShapeDtypeStruct((B, Q, PADC), jnp.float32),
            jax.ShapeDtypeStruct((B, Q, 8), jnp.float32),
        ],
    )(pred_logits, positive_map)


def _bisect_core(v, k):
    def step(_, lohi):
        lo, hi = lohi
        mid = lo + (hi - lo) // 2
        t = lax.bitcast_convert_type(mid, jnp.float32)
        cnt = jnp.sum((v >= t).astype(jnp.int32), axis=1, keepdims=True)
        ge = cnt >= k
        return jnp.where(ge, mid, lo), jnp.where(ge, hi, mid)

    lo = jnp.zeros((v.shape[0], 1), jnp.int32)
    hi = jnp.full((v.shape[0], 1), FINF_BITS, jnp.int32)
    lo, hi = lax.fori_loop(0, 31, step, (lo, hi))
    return lax.bitcast_convert_type(lo, jnp.float32)


def _k2_body(cm_ref, t_ref):
    t = _bisect_core(cm_ref[...], K)
    t_ref[...] = jnp.broadcast_to(t, (B, 16))


def _chunk_threshold(cm2):
    return pl.pallas_call(
        _k2_body,
        out_shape=jax.ShapeDtypeStruct((B, 16), jnp.float32),
    )(cm2)


def _k4_body(cand_ref, nch_ref, t_ref):
    slot = lax.broadcasted_iota(jnp.int32, (1, NCAND), 1)
    nsl = nch_ref[:, 0:1] * CHUNK
    v = jnp.where(slot < nsl, cand_ref[...], NEG)
    t = _bisect_core(v, K)
    t_ref[...] = jnp.broadcast_to(t, (B, 16))


def _cand_threshold(cand2, nch_rep):
    return pl.pallas_call(
        _k4_body,
        out_shape=jax.ShapeDtypeStruct((B, 16), jnp.float32),
    )(cand2, nch_rep)


def _sc_gather(cm2, t_rep, rows):
    mesh = plsc.VectorSubcoreMesh(core_axis_name="c", subcore_axis_name="s")

    @functools.partial(
        pl.kernel,
        mesh=mesh,
        compiler_params=pltpu.CompilerParams(needs_layout_passes=False),
        out_type=[
            jax.ShapeDtypeStruct((B, CCAP, CHUNK), jnp.float32),
            jax.ShapeDtypeStruct((B, 3, 128), jnp.int32),
            jax.ShapeDtypeStruct((B, 16), jnp.int32),
        ],
        scratch_types=[
            pltpu.VMEM((NCH,), jnp.float32),
            pltpu.VMEM((NCH,), jnp.float32),
            pltpu.VMEM((16,), jnp.float32),
            pltpu.VMEM((CCAP,), jnp.int32),
            pltpu.VMEM((3, 128), jnp.int32),
            pltpu.VMEM((3, 128), jnp.int32),
            pltpu.VMEM((16,), jnp.int32),
            pltpu.VMEM((CCAP, CHUNK), jnp.float32),
            pltpu.VMEM((CCAP, CHUNK), jnp.float32),
            pltpu.SemaphoreType.DMA,
            pltpu.SemaphoreType.DMA,
            pltpu.SemaphoreType.DMA,
            pltpu.SemaphoreType.DMA,
        ],
    )
    def k3(cm_hbm, t_hbm, rows_hbm, cand_out, cids_out, nch_out,
           cm_v0, cm_v1, t_v, ids1_v, ids2_v0, ids2_v1, nch_v,
           cand_v0, cand_v1, csem0, csem1, gsem, wsem):
        wid = lax.axis_index("s") * NC_SC + lax.axis_index("c")
        b0 = wid
        b1 = wid + NW
        cm_bufs = (cm_v0, cm_v1)
        ids2_bufs = (ids2_v0, ids2_v1)
        cand_bufs = (cand_v0, cand_v1)
        cm0 = pltpu.async_copy(cm_hbm.at[b0], cm_v0, csem0)
        cm1 = pltpu.async_copy(cm_hbm.at[b1], cm_v1, csem1)

        def compact(r, b, cm_wait):
            cm_v = cm_bufs[r]
            ids2_v = ids2_bufs[r]
            cand_v = cand_bufs[r]
            pltpu.sync_copy(t_hbm.at[b], t_v)
            tval = t_v[...]
            gbase = b * NCH

            def zbody(j, carry):
                ids1_v[pl.ds(j * 16, 16)] = jnp.zeros((16,), jnp.int32)
                return carry

            lax.fori_loop(0, CCAP // 16, zbody, jnp.int32(0))
            cm_wait.wait()

            def cbody(j, off):
                v = cm_v[pl.ds(j * 16, 16)]
                mask = v >= tval
                ids = gbase + j * 16 + lax.iota(jnp.int32, 16)
                offc = jnp.minimum(off, CCAP - 16)
                plsc.store_compressed(ids1_v.at[pl.ds(offc, 16)], ids,
                                      mask=mask)
                return off + jnp.max(plsc.all_reduce_population_count(mask))

            off = lax.fori_loop(0, NCH // 16, cbody, jnp.int32(0))
            nch_v[...] = jnp.full((16,), jnp.minimum(off, CCAP), jnp.int32)
            pltpu.sync_copy(nch_v, nch_out.at[b])
            for i in range(3):
                for j in range(8):
                    ids2_v[i, pl.ds(j * 16, 16)] = (
                        ids1_v[pl.ds(i * 128 + j * 16, 16)])
            for i in range(3):
                for j in range(8):
                    ids2_v[i, pl.ds(j * 16, 16)] = jnp.zeros((16,), jnp.int32)
            gs = [pltpu.async_copy(rows_hbm.at[ids2_v.at[i]],
                                   cand_v.at[pl.ds(i * 128, 128)], gsem)
                  for i in range(3)]
            ws = pltpu.async_copy(ids2_v, cids_out.at[b], wsem)
            return gs, ws

        g0, w0 = compact(0, b0, cm0)
        g1, w1 = compact(1, b1, cm1)
        for g in g0:
            g.wait()
        wc0 = pltpu.async_copy(cand_v0, cand_out.at[b0], wsem)
        for g in g1:
            g.wait()
        wc1 = pltpu.async_copy(cand_v1, cand_out.at[b1], wsem)
        w0.wait()
        w1.wait()
        wc0.wait()
        wc1.wait()

    return k3(cm2, t_rep, rows)


def _sc_select(cand2, cids, nch_rep, v3_rep):
    mesh = plsc.VectorSubcoreMesh(core_axis_name="c", subcore_axis_name="s")

    @functools.partial(
        pl.kernel,
        mesh=mesh,
        compiler_params=pltpu.CompilerParams(needs_layout_passes=False),
        out_type=[
            jax.ShapeDtypeStruct((B, SCAP), jnp.float32),
            jax.ShapeDtypeStruct((B, SCAP), jnp.int32),
            jax.ShapeDtypeStruct((B, 16), jnp.int32),
        ],
        scratch_types=[
            pltpu.VMEM((NCAND,), jnp.float32),
            pltpu.VMEM((NCAND,), jnp.float32),
            pltpu.VMEM((CCAP,), jnp.int32),
            pltpu.VMEM((CCAP,), jnp.int32),
            pltpu.VMEM((16,), jnp.float32),
            pltpu.VMEM((16,), jnp.int32),
            pltpu.VMEM((SCAP,), jnp.float32),
            pltpu.VMEM((SCAP,), jnp.int32),
            pltpu.VMEM((16,), jnp.int32),
            pltpu.SemaphoreType.DMA,
            pltpu.SemaphoreType.DMA,
            pltpu.SemaphoreType.DMA,
        ],
    )
    def k5(cand_hbm, cids_hbm, nch_hbm, v3_hbm, selv_out, selp_out, nsel_out,
           cand_v0, cand_v1, cid_v0, cid_v1, t_v, n_v, selv_v, selp_v, nsel_v,
           csem0, csem1, wsem):
        wid = lax.axis_index("s") * NC_SC + lax.axis_index("c")
        b0 = wid
        b1 = wid + NW
        cand_bufs = (cand_v0, cand_v1)
        cid_bufs = (cid_v0, cid_v1)
        dmas = [
            (pltpu.async_copy(cand_hbm.at[b0], cand_v0, csem0),
             pltpu.async_copy(cids_hbm.at[b0], cid_v0, csem0)),
            (pltpu.async_copy(cand_hbm.at[b1], cand_v1, csem1),
             pltpu.async_copy(cids_hbm.at[b1], cid_v1, csem1)),
        ]
        writes = []
        for r, b in ((0, b0), (1, b1)):
            cand_v = cand_bufs[r]
            cid_v = cid_bufs[r]
            pltpu.sync_copy(v3_hbm.at[b], t_v)
            pltpu.sync_copy(nch_hbm.at[b], n_v)
            tval = t_v[...]
            nch = jnp.max(n_v[...])
            nslots = nch * CHUNK
            gsub = b * NCH
            for d in dmas[r]:
                d.wait()

            def sbody(j, off):
                base = j * 16
                v = cand_v[pl.ds(base, 16)]
                s = base + lax.iota(jnp.int32, 16)
                mask = (v >= tval) & (s < nslots)
                cidg = plsc.load_gather(
                    cid_v, [lax.shift_right_logical(s, 7)])
                pfi = (cidg - gsub) * CHUNK + (s & 127)
                offc = jnp.minimum(off, SCAP - 16)
                plsc.store_compressed(selv_v.at[pl.ds(offc, 16)], v,
                                      mask=mask)
                plsc.store_compressed(selp_v.at[pl.ds(offc, 16)], pfi,
                                      mask=mask)
                return off + jnp.max(plsc.all_reduce_population_count(mask))

            nit = (nslots + 15) // 16
            off = lax.fori_loop(0, nit, sbody, jnp.int32(0))
            nsel_v[...] = jnp.full((16,), jnp.minimum(off, SCAP), jnp.int32)
            pltpu.sync_copy(nsel_v, nsel_out.at[b])
            writes.append(pltpu.async_copy(selv_v, selv_out.at[b], wsem))
            writes.append(pltpu.async_copy(selp_v, selp_out.at[b], wsem))
            if r == 0:
                for w in writes:
                    w.wait()
                writes = []
        for w in writes:
            w.wait()

    return k5(cand2, cids, nch_rep, v3_rep)


def _k6_body(selv_ref, selp_ref, selvc_ref, selpc_ref, nsel_ref, boxes_ref,
             ts_ref, sc_ref, lb_ref, bx_ref):
    v = selv_ref[0]
    p = selp_ref[0]
    nsel = jnp.max(nsel_ref[0])
    slot = lax.broadcasted_iota(jnp.int32, (1, SCAP), 1)
    v = jnp.where(slot < nsel, v, NEG)
    p = jnp.where(slot < nsel, p, jnp.int32(2**31 - 1))
    vcol = selvc_ref[0]
    pcol = selpc_ref[0]
    slotc = lax.broadcasted_iota(jnp.int32, (SCAP, 1), 0)
    vcol = jnp.where(slotc < nsel, vcol, NEG)
    pcol = jnp.where(slotc < nsel, pcol, jnp.int32(2**31 - 1))
    beats = (vcol > v) | ((vcol == v) & (pcol < p))
    beaten = (v > vcol) | ((v == vcol) & (p < pcol))
    rank_row = jnp.sum(beats.astype(jnp.int32), axis=0, keepdims=True)
    rank_col = jnp.sum(beaten.astype(jnp.int32), axis=1, keepdims=True)
    ohc = rank_col == lax.broadcasted_iota(jnp.int32, (SCAP, K), 1)
    sc_ref[0] = jnp.sum(jnp.where(ohc, vcol, 0.0), axis=0, keepdims=True)
    pf_row = jnp.sum(jnp.where(ohc, pcol, 0), axis=0, keepdims=True)
    lb_ref[0] = jnp.bitwise_and(pf_row, 1023)
    ohr = rank_row == lax.broadcasted_iota(jnp.int32, (K, SCAP), 0)
    pf_col = jnp.sum(jnp.where(ohr, p, 0), axis=1, keepdims=True)
    q_col = lax.shift_right_logical(pf_col, 10)
    oh2 = (lax.broadcasted_iota(jnp.int32, (K, Q), 1) == q_col)
    bxs = lax.dot_general(oh2.astype(jnp.float32), boxes_ref[0],
                          (((1,), (0,)), ((), ())),
                          preferred_element_type=jnp.float32,
                          precision=lax.Precision.HIGHEST)
    cx, cy, w, h = bxs[:, 0:1], bxs[:, 1:2], bxs[:, 2:3], bxs[:, 3:4]
    ts = ts_ref[0]
    ih = ts[0:1, 0:1]
    iw = ts[0:1, 1:2]
    bx_ref[0] = jnp.concatenate(
        [(cx - 0.5 * w) * iw, (cy - 0.5 * h) * ih,
         (cx + 0.5 * w) * iw, (cy + 0.5 * h) * ih], axis=1)


def _order_and_extract(selv3, selp3, selvc3, selpc3, nsel3, pred_boxes, ts3):
    return pl.pallas_call(
        _k6_body,
        grid=(B,),
        in_specs=[
            pl.BlockSpec((1, 1, SCAP), lambda b: (b, 0, 0)),
            pl.BlockSpec((1, 1, SCAP), lambda b: (b, 0, 0)),
            pl.BlockSpec((1, SCAP, 1), lambda b: (b, 0, 0)),
            pl.BlockSpec((1, SCAP, 1), lambda b: (b, 0, 0)),
            pl.BlockSpec((1, 1, 16), lambda b: (b, 0, 0)),
            pl.BlockSpec((1, Q, 4), lambda b: (b, 0, 0)),
            pl.BlockSpec((1, 1, 2), lambda b: (b, 0, 0)),
        ],
        out_specs=[
            pl.BlockSpec((1, 1, K), lambda b: (b, 0, 0)),
            pl.BlockSpec((1, 1, K), lambda b: (b, 0, 0)),
            pl.BlockSpec((1, K, 4), lambda b: (b, 0, 0)),
        ],
        out_shape=[
            jax.ShapeDtypeStruct((B, 1, K), jnp.float32),
            jax.ShapeDtypeStruct((B, 1, K), jnp.int32),
            jax.ShapeDtypeStruct((B, K, 4), jnp.float32),
        ],
    )(selv3, selp3, selvc3, selpc3, nsel3, pred_boxes, ts3)


def kernel(pred_logits, pred_boxes, target_sizes, positive_map):
    probp, cm = _score_map(pred_logits, positive_map)
    cm2 = cm.reshape(B, NCH)
    t_rep = _chunk_threshold(cm2)
    rows = probp.reshape(B * NCH, CHUNK)
    cand, cids, nch_rep = _sc_gather(cm2, t_rep, rows)
    cand2 = cand.reshape(B, NCAND)
    v3_rep = _cand_threshold(cand2, nch_rep)
    cids2 = cids.reshape(B, CCAP)
    selv, selp, nsel_rep = _sc_select(cand2, cids2, nch_rep, v3_rep)
    scores3, labels3, boxes = _order_and_extract(
        selv.reshape(B, 1, SCAP), selp.reshape(B, 1, SCAP),
        selv.reshape(B, SCAP, 1), selp.reshape(B, SCAP, 1),
        nsel_rep.reshape(B, 1, 16), pred_boxes,
        target_sizes.reshape(B, 1, 2))
    return scores3.reshape(B, K), labels3.reshape(B, K), boxes


# speedup vs baseline: 2.0468x; 2.0468x over previous
import functools

import jax
import jax.numpy as jnp
from jax import lax
from jax.experimental import pallas as pl
from jax.experimental.pallas import tpu as pltpu
from jax.experimental.pallas import tpu_sc as plsc

B, Q, T = 64, 900, 256
C = 900
K = 300
PADC = 1024
CHUNK = 64
NCH = Q * PADC // CHUNK
CCAP = 384
NCAND = CCAP * CHUNK
SCAP = 512
FINF_BITS = 0x7F800000
NC_SC, NS_SC = 2, 16
NW = NC_SC * NS_SC
NEG = -jnp.inf


def _k1_body(logits_ref, pmap_ref, probp_ref, cm_ref):
    x = logits_ref[0]
    s = jax.nn.sigmoid(x)
    p = lax.dot_general(s, pmap_ref[...], (((1,), (1,)), ((), ())),
                        preferred_element_type=jnp.float32)
    pad = jnp.full((Q, PADC - C), NEG, jnp.float32)
    probp_ref[0] = jnp.concatenate([p, pad], axis=1)
    cms = [jnp.max(p[:, 64 * i:64 * (i + 1)], axis=1, keepdims=True)
           for i in range(14)]
    cms.append(jnp.max(p[:, 896:900], axis=1, keepdims=True))
    cms.append(jnp.full((Q, 1), NEG, jnp.float32))
    cm_ref[0] = jnp.concatenate(cms, axis=1)


def _score_map(pred_logits, positive_map):
    return pl.pallas_call(
        _k1_body,
        grid=(B,),
        in_specs=[
            pl.BlockSpec((1, Q, T), lambda b: (b, 0, 0)),
            pl.BlockSpec((Q, T), lambda b: (0, 0)),
        ],
        out_specs=[
            pl.BlockSpec((1, Q, PADC), lambda b: (b, 0, 0)),
            pl.BlockSpec((1, Q, 16), lambda b: (b, 0, 0)),
        ],
        out_shape=[
            jax.ShapeDtypeStruct((B, Q, PADC), jnp.float32),
            jax.ShapeDtypeStruct((B, Q, 16), jnp.float32),
        ],
    )(pred_logits, positive_map)


def _bisect_core(v, k):
    def step(_, lohi):
        lo, hi = lohi
        mid = lo + (hi - lo) // 2
        t = lax.bitcast_convert_type(mid, jnp.float32)
        cnt = jnp.sum((v >= t).astype(jnp.int32), axis=1, keepdims=True)
        ge = cnt >= k
        return jnp.where(ge, mid, lo), jnp.where(ge, hi, mid)

    lo = jnp.zeros((v.shape[0], 1), jnp.int32)
    hi = jnp.full((v.shape[0], 1), FINF_BITS, jnp.int32)
    lo, hi = lax.fori_loop(0, 31, step, (lo, hi))
    return lax.bitcast_convert_type(lo, jnp.float32)


def _k2_body(cm_ref, t_ref):
    t = _bisect_core(cm_ref[...], K)
    t_ref[...] = jnp.broadcast_to(t, (B, 16))


def _chunk_threshold(cm2):
    return pl.pallas_call(
        _k2_body,
        out_shape=jax.ShapeDtypeStruct((B, 16), jnp.float32),
    )(cm2)


def _k4_body(cand_ref, nch_ref, t_ref):
    slot = lax.broadcasted_iota(jnp.int32, (1, NCAND), 1)
    nsl = nch_ref[:, 0:1] * CHUNK
    v = jnp.where(slot < nsl, cand_ref[...], NEG)
    t = _bisect_core(v, K)
    t_ref[...] = jnp.broadcast_to(t, (B, 16))


def _cand_threshold(cand2, nch_rep):
    return pl.pallas_call(
        _k4_body,
        out_shape=jax.ShapeDtypeStruct((B, 16), jnp.float32),
    )(cand2, nch_rep)


def _sc_gather(cm2, t_rep, rows):
    mesh = plsc.VectorSubcoreMesh(core_axis_name="c", subcore_axis_name="s")

    @functools.partial(
        pl.kernel,
        mesh=mesh,
        compiler_params=pltpu.CompilerParams(
            needs_layout_passes=False, use_tc_tiling_on_sc=False),
        out_type=[
            jax.ShapeDtypeStruct((B, CCAP, CHUNK), jnp.float32),
            jax.ShapeDtypeStruct((B, 3, 128), jnp.int32),
            jax.ShapeDtypeStruct((B, 16), jnp.int32),
        ],
        scratch_types=[
            pltpu.VMEM((NCH,), jnp.float32),
            pltpu.VMEM((NCH,), jnp.float32),
            pltpu.VMEM((16,), jnp.float32),
            pltpu.VMEM((CCAP,), jnp.int32),
            pltpu.VMEM((3, 128), jnp.int32),
            pltpu.VMEM((3, 128), jnp.int32),
            pltpu.VMEM((16,), jnp.int32),
            pltpu.VMEM((CCAP, CHUNK), jnp.float32),
            pltpu.VMEM((CCAP, CHUNK), jnp.float32),
            pltpu.SemaphoreType.DMA,
            pltpu.SemaphoreType.DMA,
            pltpu.SemaphoreType.DMA,
            pltpu.SemaphoreType.DMA,
        ],
    )
    def k3(cm_hbm, t_hbm, rows_hbm, cand_out, cids_out, nch_out,
           cm_v0, cm_v1, t_v, ids1_v, ids2_v0, ids2_v1, nch_v,
           cand_v0, cand_v1, csem0, csem1, gsem, wsem):
        wid = lax.axis_index("s") * NC_SC + lax.axis_index("c")
        b0 = wid
        b1 = wid + NW
        cm_bufs = (cm_v0, cm_v1)
        ids2_bufs = (ids2_v0, ids2_v1)
        cand_bufs = (cand_v0, cand_v1)
        cm0 = pltpu.async_copy(cm_hbm.at[b0], cm_v0, csem0)
        cm1 = pltpu.async_copy(cm_hbm.at[b1], cm_v1, csem1)

        def compact(r, b, cm_wait):
            cm_v = cm_bufs[r]
            ids2_v = ids2_bufs[r]
            cand_v = cand_bufs[r]
            pltpu.sync_copy(t_hbm.at[b], t_v)
            tval = t_v[...]
            gbase = b * NCH

            def zbody(j, carry):
                ids1_v[pl.ds(j * 16, 16)] = jnp.zeros((16,), jnp.int32)
                return carry

            lax.fori_loop(0, CCAP // 16, zbody, jnp.int32(0))
            cm_wait.wait()

            def cbody(j, off):
                v = cm_v[pl.ds(j * 16, 16)]
                mask = v >= tval
                ids = gbase + j * 16 + lax.iota(jnp.int32, 16)
                offc = jnp.minimum(off, CCAP - 16)
                plsc.store_compressed(ids1_v.at[pl.ds(offc, 16)], ids,
                                      mask=mask)
                return off + jnp.max(plsc.all_reduce_population_count(mask))

            off = lax.fori_loop(0, NCH // 16, cbody, jnp.int32(0))
            nch_v[...] = jnp.full((16,), jnp.minimum(off, CCAP), jnp.int32)
            pltpu.sync_copy(nch_v, nch_out.at[b])
            for i in range(3):
                for j in range(8):
                    ids2_v[i, pl.ds(j * 16, 16)] = (
                        ids1_v[pl.ds(i * 128 + j * 16, 16)])
            gs = [pltpu.async_copy(rows_hbm.at[ids2_v.at[i]],
                                   cand_v.at[pl.ds(i * 128, 128)], gsem)
                  for i in range(3)]
            ws = pltpu.async_copy(ids2_v, cids_out.at[b], wsem)
            return gs, ws

        g0, w0 = compact(0, b0, cm0)
        g1, w1 = compact(1, b1, cm1)
        for g in g0:
            g.wait()
        wc0 = pltpu.async_copy(cand_v0, cand_out.at[b0], wsem)
        for g in g1:
            g.wait()
        wc1 = pltpu.async_copy(cand_v1, cand_out.at[b1], wsem)
        w0.wait()
        w1.wait()
        wc0.wait()
        wc1.wait()

    return k3(cm2, t_rep, rows)


def _sc_select(cand2, cids, nch_rep, v3_rep):
    mesh = plsc.VectorSubcoreMesh(core_axis_name="c", subcore_axis_name="s")

    @functools.partial(
        pl.kernel,
        mesh=mesh,
        compiler_params=pltpu.CompilerParams(
            needs_layout_passes=False, use_tc_tiling_on_sc=False),
        out_type=[
            jax.ShapeDtypeStruct((B, SCAP), jnp.float32),
            jax.ShapeDtypeStruct((B, SCAP), jnp.int32),
            jax.ShapeDtypeStruct((B, 16), jnp.int32),
        ],
        scratch_types=[
            pltpu.VMEM((NCAND,), jnp.float32),
            pltpu.VMEM((NCAND,), jnp.float32),
            pltpu.VMEM((CCAP,), jnp.int32),
            pltpu.VMEM((CCAP,), jnp.int32),
            pltpu.VMEM((16,), jnp.float32),
            pltpu.VMEM((16,), jnp.int32),
            pltpu.VMEM((SCAP,), jnp.float32),
            pltpu.VMEM((SCAP,), jnp.int32),
            pltpu.VMEM((16,), jnp.int32),
            pltpu.SemaphoreType.DMA,
            pltpu.SemaphoreType.DMA,
            pltpu.SemaphoreType.DMA,
        ],
    )
    def k5(cand_hbm, cids_hbm, nch_hbm, v3_hbm, selv_out, selp_out, nsel_out,
           cand_v0, cand_v1, cid_v0, cid_v1, t_v, n_v, selv_v, selp_v, nsel_v,
           csem0, csem1, wsem):
        wid = lax.axis_index("s") * NC_SC + lax.axis_index("c")
        b0 = wid
        b1 = wid + NW
        cand_bufs = (cand_v0, cand_v1)
        cid_bufs = (cid_v0, cid_v1)
        dmas = [
            (pltpu.async_copy(cand_hbm.at[b0], cand_v0, csem0),
             pltpu.async_copy(cids_hbm.at[b0], cid_v0, csem0)),
            (pltpu.async_copy(cand_hbm.at[b1], cand_v1, csem1),
             pltpu.async_copy(cids_hbm.at[b1], cid_v1, csem1)),
        ]
        writes = []
        for r, b in ((0, b0), (1, b1)):
            cand_v = cand_bufs[r]
            cid_v = cid_bufs[r]
            pltpu.sync_copy(v3_hbm.at[b], t_v)
            pltpu.sync_copy(nch_hbm.at[b], n_v)
            tval = t_v[...]
            nch = jnp.max(n_v[...])
            nslots = nch * CHUNK
            gsub = b * NCH
            for d in dmas[r]:
                d.wait()

            def sbody(j, off):
                base = j * 16
                v = cand_v[pl.ds(base, 16)]
                s = base + lax.iota(jnp.int32, 16)
                mask = (v >= tval) & (s < nslots)
                cidg = plsc.load_gather(
                    cid_v, [lax.shift_right_logical(s, 6)])
                pfi = (cidg - gsub) * CHUNK + (s & 63)
                offc = jnp.minimum(off, SCAP - 16)
                plsc.store_compressed(selv_v.at[pl.ds(offc, 16)], v,
                                      mask=mask)
                plsc.store_compressed(selp_v.at[pl.ds(offc, 16)], pfi,
                                      mask=mask)
                return off + jnp.max(plsc.all_reduce_population_count(mask))

            nit = (nslots + 15) // 16
            off = lax.fori_loop(0, nit, sbody, jnp.int32(0))
            nsel_v[...] = jnp.full((16,), jnp.minimum(off, SCAP), jnp.int32)
            pltpu.sync_copy(nsel_v, nsel_out.at[b])
            writes.append(pltpu.async_copy(selv_v, selv_out.at[b], wsem))
            writes.append(pltpu.async_copy(selp_v, selp_out.at[b], wsem))
            if r == 0:
                for w in writes:
                    w.wait()
                writes = []
        for w in writes:
            w.wait()

    return k5(cand2, cids, nch_rep, v3_rep)


def _k6_body(selv_ref, selp_ref, selvc_ref, selpc_ref, nsel_ref, boxes_ref,
             ts_ref, sc_ref, lb_ref, bx_ref):
    v = selv_ref[0]
    p = selp_ref[0]
    nsel = jnp.max(nsel_ref[0])
    slot = lax.broadcasted_iota(jnp.int32, (1, SCAP), 1)
    v = jnp.where(slot < nsel, v, NEG)
    p = jnp.where(slot < nsel, p, jnp.int32(2**31 - 1))
    vcol = selvc_ref[0]
    pcol = selpc_ref[0]
    slotc = lax.broadcasted_iota(jnp.int32, (SCAP, 1), 0)
    vcol = jnp.where(slotc < nsel, vcol, NEG)
    pcol = jnp.where(slotc < nsel, pcol, jnp.int32(2**31 - 1))
    beats = (vcol > v) | ((vcol == v) & (pcol < p))
    beaten = (v > vcol) | ((v == vcol) & (p < pcol))
    rank_row = jnp.sum(beats.astype(jnp.int32), axis=0, keepdims=True)
    rank_col = jnp.sum(beaten.astype(jnp.int32), axis=1, keepdims=True)
    ohc = rank_col == lax.broadcasted_iota(jnp.int32, (SCAP, K), 1)
    sc_ref[0] = jnp.sum(jnp.where(ohc, vcol, 0.0), axis=0, keepdims=True)
    pf_row = jnp.sum(jnp.where(ohc, pcol, 0), axis=0, keepdims=True)
    lb_ref[0] = jnp.bitwise_and(pf_row, 1023)
    ohr = rank_row == lax.broadcasted_iota(jnp.int32, (K, SCAP), 0)
    pf_col = jnp.sum(jnp.where(ohr, p, 0), axis=1, keepdims=True)
    q_col = lax.shift_right_logical(pf_col, 10)
    oh2 = (lax.broadcasted_iota(jnp.int32, (K, Q), 1) == q_col)
    bxs = lax.dot_general(oh2.astype(jnp.float32), boxes_ref[0],
                          (((1,), (0,)), ((), ())),
                          preferred_element_type=jnp.float32,
                          precision=lax.Precision.HIGHEST)
    cx, cy, w, h = bxs[:, 0:1], bxs[:, 1:2], bxs[:, 2:3], bxs[:, 3:4]
    ts = ts_ref[0]
    ih = ts[0:1, 0:1]
    iw = ts[0:1, 1:2]
    bx_ref[0] = jnp.concatenate(
        [(cx - 0.5 * w) * iw, (cy - 0.5 * h) * ih,
         (cx + 0.5 * w) * iw, (cy + 0.5 * h) * ih], axis=1)


def _order_and_extract(selv3, selp3, selvc3, selpc3, nsel3, pred_boxes, ts3):
    return pl.pallas_call(
        _k6_body,
        grid=(B,),
        in_specs=[
            pl.BlockSpec((1, 1, SCAP), lambda b: (b, 0, 0)),
            pl.BlockSpec((1, 1, SCAP), lambda b: (b, 0, 0)),
            pl.BlockSpec((1, SCAP, 1), lambda b: (b, 0, 0)),
            pl.BlockSpec((1, SCAP, 1), lambda b: (b, 0, 0)),
            pl.BlockSpec((1, 1, 16), lambda b: (b, 0, 0)),
            pl.BlockSpec((1, Q, 4), lambda b: (b, 0, 0)),
            pl.BlockSpec((1, 1, 2), lambda b: (b, 0, 0)),
        ],
        out_specs=[
            pl.BlockSpec((1, 1, K), lambda b: (b, 0, 0)),
            pl.BlockSpec((1, 1, K), lambda b: (b, 0, 0)),
            pl.BlockSpec((1, K, 4), lambda b: (b, 0, 0)),
        ],
        out_shape=[
            jax.ShapeDtypeStruct((B, 1, K), jnp.float32),
            jax.ShapeDtypeStruct((B, 1, K), jnp.int32),
            jax.ShapeDtypeStruct((B, K, 4), jnp.float32),
        ],
    )(selv3, selp3, selvc3, selpc3, nsel3, pred_boxes, ts3)


def kernel(pred_logits, pred_boxes, target_sizes, positive_map):
    probp, cm = _score_map(pred_logits, positive_map)
    cm2 = cm.reshape(B, NCH)
    t_rep = _chunk_threshold(cm2)
    rows = probp.reshape(B * NCH, CHUNK)
    cand, cids, nch_rep = _sc_gather(cm2, t_rep, rows)
    cand2 = cand.reshape(B, NCAND)
    v3_rep = _cand_threshold(cand2, nch_rep)
    cids2 = cids.reshape(B, CCAP)
    selv, selp, nsel_rep = _sc_select(cand2, cids2, nch_rep, v3_rep)
    scores3, labels3, boxes = _order_and_extract(
        selv.reshape(B, 1, SCAP), selp.reshape(B, 1, SCAP),
        selv.reshape(B, SCAP, 1), selp.reshape(B, SCAP, 1),
        nsel_rep.reshape(B, 1, 16), pred_boxes,
        target_sizes.reshape(B, 1, 2))
    return scores3.reshape(B, K), labels3.reshape(B, K), boxes


# speedup vs baseline: 2.1474x; 1.0492x over previous
import functools

import jax
import jax.numpy as jnp
from jax import lax
from jax.experimental import pallas as pl
from jax.experimental.pallas import tpu as pltpu
from jax.experimental.pallas import tpu_sc as plsc

B, Q, T = 64, 900, 256
C = 900
K = 300
PADC = 1024
CHUNK = 64
NCH = Q * PADC // CHUNK
CCAP = 384
NCAND = CCAP * CHUNK
SCAP = 512
FINF_BITS = 0x7F800000
NC_SC, NS_SC = 2, 16
NW = NC_SC * NS_SC
NEG = -jnp.inf
SB = 8


def _k1_body(logits_ref, pmap_ref, probp_ref, cm_ref):
    x = logits_ref[0]
    s = jax.nn.sigmoid(x)
    p = lax.dot_general(s, pmap_ref[...], (((1,), (1,)), ((), ())),
                        preferred_element_type=jnp.float32)
    pad = jnp.full((Q, PADC - C), NEG, jnp.float32)
    probp_ref[0] = jnp.concatenate([p, pad], axis=1)
    cms = [jnp.max(p[:, 64 * i:64 * (i + 1)], axis=1, keepdims=True)
           for i in range(14)]
    cms.append(jnp.max(p[:, 896:900], axis=1, keepdims=True))
    cms.append(jnp.full((Q, 1), NEG, jnp.float32))
    cm_ref[0] = jnp.concatenate(cms, axis=1)


def _score_map(pred_logits, positive_map):
    return pl.pallas_call(
        _k1_body,
        grid=(B,),
        in_specs=[
            pl.BlockSpec((1, Q, T), lambda b: (b, 0, 0)),
            pl.BlockSpec((Q, T), lambda b: (0, 0)),
        ],
        out_specs=[
            pl.BlockSpec((1, Q, PADC), lambda b: (b, 0, 0)),
            pl.BlockSpec((1, Q, 16), lambda b: (b, 0, 0)),
        ],
        out_shape=[
            jax.ShapeDtypeStruct((B, Q, PADC), jnp.float32),
            jax.ShapeDtypeStruct((B, Q, 16), jnp.float32),
        ],
    )(pred_logits, positive_map)


def _bisect_core(v, k):
    def step(_, lohi):
        lo, hi = lohi
        mid = lo + (hi - lo) // 2
        t = lax.bitcast_convert_type(mid, jnp.float32)
        cnt = jnp.sum((v >= t).astype(jnp.int32), axis=1, keepdims=True)
        ge = cnt >= k
        return jnp.where(ge, mid, lo), jnp.where(ge, hi, mid)

    lo = jnp.zeros((v.shape[0], 1), jnp.int32)
    hi = jnp.full((v.shape[0], 1), FINF_BITS, jnp.int32)
    lo, hi = lax.fori_loop(0, 31, step, (lo, hi))
    return lax.bitcast_convert_type(lo, jnp.float32)


def _k2_body(cm_ref, t_ref):
    t = _bisect_core(cm_ref[...], K)
    t_ref[...] = jnp.broadcast_to(t, (B, 16))


def _chunk_threshold(cm2):
    return pl.pallas_call(
        _k2_body,
        out_shape=jax.ShapeDtypeStruct((B, 16), jnp.float32),
    )(cm2)


def _k4_body(cand_ref, nch_ref, t_ref):
    slot = lax.broadcasted_iota(jnp.int32, (1, NCAND), 1)
    nsl = nch_ref[:, 0:1] * CHUNK
    v = jnp.where(slot < nsl, cand_ref[...], NEG)
    t = _bisect_core(v, K)
    t_ref[...] = jnp.broadcast_to(t, (B, 16))


def _cand_threshold(cand2, nch_rep):
    return pl.pallas_call(
        _k4_body,
        out_shape=jax.ShapeDtypeStruct((B, 16), jnp.float32),
    )(cand2, nch_rep)


def _sc_gather(cm2, t_rep, rows):
    mesh = plsc.VectorSubcoreMesh(core_axis_name="c", subcore_axis_name="s")

    @functools.partial(
        pl.kernel,
        mesh=mesh,
        compiler_params=pltpu.CompilerParams(
            needs_layout_passes=False, use_tc_tiling_on_sc=False),
        out_type=[
            jax.ShapeDtypeStruct((B, CCAP, CHUNK), jnp.float32),
            jax.ShapeDtypeStruct((B, 3, 128), jnp.int32),
            jax.ShapeDtypeStruct((B, 16), jnp.int32),
        ],
        scratch_types=[
            pltpu.VMEM((NCH,), jnp.float32),
            pltpu.VMEM((NCH,), jnp.float32),
            pltpu.VMEM((16,), jnp.float32),
            pltpu.VMEM((CCAP,), jnp.int32),
            pltpu.VMEM((3, 128), jnp.int32),
            pltpu.VMEM((3, 128), jnp.int32),
            pltpu.VMEM((16,), jnp.int32),
            pltpu.VMEM((CCAP, CHUNK), jnp.float32),
            pltpu.VMEM((CCAP, CHUNK), jnp.float32),
            pltpu.SemaphoreType.DMA,
            pltpu.SemaphoreType.DMA,
            pltpu.SemaphoreType.DMA,
            pltpu.SemaphoreType.DMA,
        ],
    )
    def k3(cm_hbm, t_hbm, rows_hbm, cand_out, cids_out, nch_out,
           cm_v0, cm_v1, t_v, ids1_v, ids2_v0, ids2_v1, nch_v,
           cand_v0, cand_v1, csem0, csem1, gsem, wsem):
        wid = lax.axis_index("s") * NC_SC + lax.axis_index("c")
        b0 = wid
        b1 = wid + NW
        cm_bufs = (cm_v0, cm_v1)
        ids2_bufs = (ids2_v0, ids2_v1)
        cand_bufs = (cand_v0, cand_v1)
        cm0 = pltpu.async_copy(cm_hbm.at[b0], cm_v0, csem0)
        cm1 = pltpu.async_copy(cm_hbm.at[b1], cm_v1, csem1)

        def compact(r, b, cm_wait):
            cm_v = cm_bufs[r]
            ids2_v = ids2_bufs[r]
            cand_v = cand_bufs[r]
            pltpu.sync_copy(t_hbm.at[b], t_v)
            tval = t_v[...]
            gbase = b * NCH

            def zbody(j, carry):
                ids1_v[pl.ds(j * 16, 16)] = jnp.zeros((16,), jnp.int32)
                return carry

            lax.fori_loop(0, CCAP // 16, zbody, jnp.int32(0))
            cm_wait.wait()

            def cbody(j, off):
                v = cm_v[pl.ds(j * 16, 16)]
                mask = v >= tval
                ids = gbase + j * 16 + lax.iota(jnp.int32, 16)
                offc = jnp.minimum(off, CCAP - 16)
                plsc.store_compressed(ids1_v.at[pl.ds(offc, 16)], ids,
                                      mask=mask)
                return off + jnp.max(plsc.all_reduce_population_count(mask))

            off = lax.fori_loop(0, NCH // 16, cbody, jnp.int32(0))
            nch_v[...] = jnp.full((16,), jnp.minimum(off, CCAP), jnp.int32)
            pltpu.sync_copy(nch_v, nch_out.at[b])
            for i in range(3):
                for j in range(8):
                    ids2_v[i, pl.ds(j * 16, 16)] = (
                        ids1_v[pl.ds(i * 128 + j * 16, 16)])
            gs = [pltpu.async_copy(rows_hbm.at[ids2_v.at[i]],
                                   cand_v.at[pl.ds(i * 128, 128)], gsem)
                  for i in range(3)]
            ws = pltpu.async_copy(ids2_v, cids_out.at[b], wsem)
            return gs, ws

        g0, w0 = compact(0, b0, cm0)
        g1, w1 = compact(1, b1, cm1)
        for g in g0:
            g.wait()
        wc0 = pltpu.async_copy(cand_v0, cand_out.at[b0], wsem)
        for g in g1:
            g.wait()
        wc1 = pltpu.async_copy(cand_v1, cand_out.at[b1], wsem)
        w0.wait()
        w1.wait()
        wc0.wait()
        wc1.wait()

    return k3(cm2, t_rep, rows)


def _sc_select(cand2, cids, nch_rep, v3_rep):
    mesh = plsc.VectorSubcoreMesh(core_axis_name="c", subcore_axis_name="s")

    @functools.partial(
        pl.kernel,
        mesh=mesh,
        compiler_params=pltpu.CompilerParams(
            needs_layout_passes=False, use_tc_tiling_on_sc=False),
        out_type=[
            jax.ShapeDtypeStruct((B, SCAP), jnp.float32),
            jax.ShapeDtypeStruct((B, SCAP), jnp.int32),
            jax.ShapeDtypeStruct((B, 16), jnp.int32),
        ],
        scratch_types=[
            pltpu.VMEM((NCAND,), jnp.float32),
            pltpu.VMEM((NCAND,), jnp.float32),
            pltpu.VMEM((CCAP,), jnp.int32),
            pltpu.VMEM((CCAP,), jnp.int32),
            pltpu.VMEM((16,), jnp.float32),
            pltpu.VMEM((16,), jnp.int32),
            pltpu.VMEM((SCAP,), jnp.float32),
            pltpu.VMEM((SCAP,), jnp.int32),
            pltpu.VMEM((16,), jnp.int32),
            pltpu.SemaphoreType.DMA,
            pltpu.SemaphoreType.DMA,
            pltpu.SemaphoreType.DMA,
        ],
    )
    def k5(cand_hbm, cids_hbm, nch_hbm, v3_hbm, selv_out, selp_out, nsel_out,
           cand_v0, cand_v1, cid_v0, cid_v1, t_v, n_v, selv_v, selp_v, nsel_v,
           csem0, csem1, wsem):
        wid = lax.axis_index("s") * NC_SC + lax.axis_index("c")
        b0 = wid
        b1 = wid + NW
        cand_bufs = (cand_v0, cand_v1)
        cid_bufs = (cid_v0, cid_v1)
        dmas = [
            (pltpu.async_copy(cand_hbm.at[b0], cand_v0, csem0),
             pltpu.async_copy(cids_hbm.at[b0], cid_v0, csem0)),
            (pltpu.async_copy(cand_hbm.at[b1], cand_v1, csem1),
             pltpu.async_copy(cids_hbm.at[b1], cid_v1, csem1)),
        ]
        writes = []
        for r, b in ((0, b0), (1, b1)):
            cand_v = cand_bufs[r]
            cid_v = cid_bufs[r]
            pltpu.sync_copy(v3_hbm.at[b], t_v)
            pltpu.sync_copy(nch_hbm.at[b], n_v)
            tval = t_v[...]
            nch = jnp.max(n_v[...])
            nslots = nch * CHUNK
            gsub = b * NCH
            for d in dmas[r]:
                d.wait()

            def sbody(j, off):
                base = j * 16
                v = cand_v[pl.ds(base, 16)]
                s = base + lax.iota(jnp.int32, 16)
                mask = (v >= tval) & (s < nslots)
                cidg = plsc.load_gather(
                    cid_v, [lax.shift_right_logical(s, 6)])
                pfi = (cidg - gsub) * CHUNK + (s & 63)
                offc = jnp.minimum(off, SCAP - 16)
                plsc.store_compressed(selv_v.at[pl.ds(offc, 16)], v,
                                      mask=mask)
                plsc.store_compressed(selp_v.at[pl.ds(offc, 16)], pfi,
                                      mask=mask)
                return off + jnp.max(plsc.all_reduce_population_count(mask))

            nit = (nslots + 15) // 16
            off = lax.fori_loop(0, nit, sbody, jnp.int32(0))
            nsel_v[...] = jnp.full((16,), jnp.minimum(off, SCAP), jnp.int32)
            pltpu.sync_copy(nsel_v, nsel_out.at[b])
            writes.append(pltpu.async_copy(selv_v, selv_out.at[b], wsem))
            writes.append(pltpu.async_copy(selp_v, selp_out.at[b], wsem))
            if r == 0:
                for w in writes:
                    w.wait()
                writes = []
        for w in writes:
            w.wait()

    return k5(cand2, cids, nch_rep, v3_rep)


def _k6_body(selv_ref, selp_ref, selvc_ref, selpc_ref, nsel_ref, boxes_ref,
             ts_ref, sc_ref, lb_ref, bx_ref):
    v = selv_ref[...]
    p = selp_ref[...]
    nsel = nsel_ref[:, :, 0:1]
    slot = lax.broadcasted_iota(jnp.int32, (SB, 1, SCAP), 2)
    v = jnp.where(slot < nsel, v, NEG)
    p = jnp.where(slot < nsel, p, jnp.int32(2**31 - 1))
    vcol = selvc_ref[...]
    pcol = selpc_ref[...]
    slotc = lax.broadcasted_iota(jnp.int32, (SB, SCAP, 1), 1)
    vcol = jnp.where(slotc < nsel, vcol, NEG)
    pcol = jnp.where(slotc < nsel, pcol, jnp.int32(2**31 - 1))
    beats = (vcol > v) | ((vcol == v) & (pcol < p))
    beaten = (v > vcol) | ((v == vcol) & (p < pcol))
    rank_row = jnp.sum(beats.astype(jnp.int32), axis=1, keepdims=True)
    rank_col = jnp.sum(beaten.astype(jnp.int32), axis=2, keepdims=True)
    ohc = rank_col == lax.broadcasted_iota(jnp.int32, (SB, SCAP, K), 2)
    sc_ref[...] = jnp.sum(jnp.where(ohc, vcol, 0.0), axis=1, keepdims=True)
    pf_row = jnp.sum(jnp.where(ohc, pcol, 0), axis=1, keepdims=True)
    lb_ref[...] = jnp.bitwise_and(pf_row, 1023)
    ohr = rank_row == lax.broadcasted_iota(jnp.int32, (SB, K, SCAP), 1)
    pf_col = jnp.sum(jnp.where(ohr, p, 0), axis=2, keepdims=True)
    q_col = lax.shift_right_logical(pf_col, 10)
    oh2 = (lax.broadcasted_iota(jnp.int32, (SB, K, Q), 2) == q_col)
    bxs = lax.dot_general(oh2.astype(jnp.float32), boxes_ref[...],
                          (((2,), (1,)), ((0,), (0,))),
                          preferred_element_type=jnp.float32,
                          precision=lax.Precision.HIGHEST)
    cx = bxs[:, :, 0:1]
    cy = bxs[:, :, 1:2]
    w = bxs[:, :, 2:3]
    h = bxs[:, :, 3:4]
    ts = ts_ref[...]
    ih = ts[:, :, 0:1]
    iw = ts[:, :, 1:2]
    bx_ref[...] = jnp.concatenate(
        [(cx - 0.5 * w) * iw, (cy - 0.5 * h) * ih,
         (cx + 0.5 * w) * iw, (cy + 0.5 * h) * ih], axis=2)


def _order_and_extract(selv3, selp3, selvc3, selpc3, nsel3, pred_boxes, ts3):
    return pl.pallas_call(
        _k6_body,
        grid=(B // SB,),
        in_specs=[
            pl.BlockSpec((SB, 1, SCAP), lambda b: (b, 0, 0)),
            pl.BlockSpec((SB, 1, SCAP), lambda b: (b, 0, 0)),
            pl.BlockSpec((SB, SCAP, 1), lambda b: (b, 0, 0)),
            pl.BlockSpec((SB, SCAP, 1), lambda b: (b, 0, 0)),
            pl.BlockSpec((SB, 1, 16), lambda b: (b, 0, 0)),
            pl.BlockSpec((SB, Q, 4), lambda b: (b, 0, 0)),
            pl.BlockSpec((SB, 1, 2), lambda b: (b, 0, 0)),
        ],
        out_specs=[
            pl.BlockSpec((SB, 1, K), lambda b: (b, 0, 0)),
            pl.BlockSpec((SB, 1, K), lambda b: (b, 0, 0)),
            pl.BlockSpec((SB, K, 4), lambda b: (b, 0, 0)),
        ],
        out_shape=[
            jax.ShapeDtypeStruct((B, 1, K), jnp.float32),
            jax.ShapeDtypeStruct((B, 1, K), jnp.int32),
            jax.ShapeDtypeStruct((B, K, 4), jnp.float32),
        ],
    )(selv3, selp3, selvc3, selpc3, nsel3, pred_boxes, ts3)


def kernel(pred_logits, pred_boxes, target_sizes, positive_map):
    probp, cm = _score_map(pred_logits, positive_map)
    cm2 = cm.reshape(B, NCH)
    t_rep = _chunk_threshold(cm2)
    rows = probp.reshape(B * NCH, CHUNK)
    cand, cids, nch_rep = _sc_gather(cm2, t_rep, rows)
    cand2 = cand.reshape(B, NCAND)
    v3_rep = _cand_threshold(cand2, nch_rep)
    cids2 = cids.reshape(B, CCAP)
    selv, selp, nsel_rep = _sc_select(cand2, cids2, nch_rep, v3_rep)
    scores3, labels3, boxes = _order_and_extract(
        selv.reshape(B, 1, SCAP), selp.reshape(B, 1, SCAP),
        selv.reshape(B, SCAP, 1), selp.reshape(B, SCAP, 1),
        nsel_rep.reshape(B, 1, 16), pred_boxes,
        target_sizes.reshape(B, 1, 2))
    return scores3.reshape(B, K), labels3.reshape(B, K), boxes


# speedup vs baseline: 2.3558x; 1.0971x over previous
import functools

import jax
import jax.numpy as jnp
from jax import lax
from jax.experimental import pallas as pl
from jax.experimental.pallas import tpu as pltpu
from jax.experimental.pallas import tpu_sc as plsc

B, Q, T = 64, 900, 256
C = 900
K = 300
PADC = 1024
CHUNK = 64
NCH = Q * PADC // CHUNK
CCAP = 384
NCAND = CCAP * CHUNK
SCAP = 512
FINF_BITS = 0x7F800000
NC_SC, NS_SC = 2, 16
NW = NC_SC * NS_SC
NEG = -jnp.inf
SB = 8


def _k1_body(logits_ref, pmap_ref, probp_ref, cm_ref):
    x = logits_ref[0]
    s = jax.nn.sigmoid(x)
    p = lax.dot_general(s, pmap_ref[...], (((1,), (1,)), ((), ())),
                        preferred_element_type=jnp.float32)
    pad = jnp.full((Q, PADC - C), NEG, jnp.float32)
    probp_ref[0] = jnp.concatenate([p, pad], axis=1)
    cms = [jnp.max(p[:, 64 * i:64 * (i + 1)], axis=1, keepdims=True)
           for i in range(14)]
    cms.append(jnp.max(p[:, 896:900], axis=1, keepdims=True))
    cms.append(jnp.full((Q, 1), NEG, jnp.float32))
    cm_ref[0] = jnp.concatenate(cms, axis=1)


def _score_map(pred_logits, positive_map):
    return pl.pallas_call(
        _k1_body,
        grid=(B,),
        in_specs=[
            pl.BlockSpec((1, Q, T), lambda b: (b, 0, 0)),
            pl.BlockSpec((Q, T), lambda b: (0, 0)),
        ],
        out_specs=[
            pl.BlockSpec((1, Q, PADC), lambda b: (b, 0, 0)),
            pl.BlockSpec((1, Q, 16), lambda b: (b, 0, 0)),
        ],
        out_shape=[
            jax.ShapeDtypeStruct((B, Q, PADC), jnp.float32),
            jax.ShapeDtypeStruct((B, Q, 16), jnp.float32),
        ],
    )(pred_logits, positive_map)


def _bisect_core(v, k):
    def step(_, lohi):
        lo, hi = lohi
        mid = lo + (hi - lo) // 2
        t = lax.bitcast_convert_type(mid, jnp.float32)
        cnt = jnp.sum((v >= t).astype(jnp.int32), axis=1, keepdims=True)
        ge = cnt >= k
        return jnp.where(ge, mid, lo), jnp.where(ge, hi, mid)

    lo = jnp.zeros((v.shape[0], 1), jnp.int32)
    hi = jnp.full((v.shape[0], 1), FINF_BITS, jnp.int32)
    lo, hi = lax.fori_loop(0, 31, step, (lo, hi))
    return lax.bitcast_convert_type(lo, jnp.float32)


def _k2_body(cm_ref, t_ref):
    t = _bisect_core(cm_ref[...], K)
    t_ref[...] = jnp.broadcast_to(t, (B, 16))


def _chunk_threshold(cm2):
    return pl.pallas_call(
        _k2_body,
        out_shape=jax.ShapeDtypeStruct((B, 16), jnp.float32),
    )(cm2)


def _k4_body(cand_ref, nch_ref, t_ref):
    slot = lax.broadcasted_iota(jnp.int32, (1, NCAND), 1)
    nsl = nch_ref[:, 0:1] * CHUNK
    v = jnp.where(slot < nsl, cand_ref[...], NEG)
    t = _bisect_core(v, K)
    t_ref[...] = jnp.broadcast_to(t, (B, 16))


def _cand_threshold(cand2, nch_rep):
    return pl.pallas_call(
        _k4_body,
        out_shape=jax.ShapeDtypeStruct((B, 16), jnp.float32),
    )(cand2, nch_rep)


def _sc_gather_select(cm2, t_rep, rows):
    mesh = plsc.VectorSubcoreMesh(core_axis_name="c", subcore_axis_name="s")

    @functools.partial(
        pl.kernel,
        mesh=mesh,
        compiler_params=pltpu.CompilerParams(
            needs_layout_passes=False, use_tc_tiling_on_sc=False),
        out_type=[
            jax.ShapeDtypeStruct((B, SCAP), jnp.float32),
            jax.ShapeDtypeStruct((B, SCAP), jnp.int32),
            jax.ShapeDtypeStruct((B, 16), jnp.int32),
        ],
        scratch_types=[
            pltpu.VMEM((NCH,), jnp.float32),
            pltpu.VMEM((NCH,), jnp.float32),
            pltpu.VMEM((16,), jnp.float32),
            pltpu.VMEM((16,), jnp.float32),
            pltpu.VMEM((CCAP,), jnp.int32),
            pltpu.VMEM((3, 128), jnp.int32),
            pltpu.VMEM((3, 128), jnp.int32),
            pltpu.VMEM((16,), jnp.int32),
            pltpu.VMEM((CCAP, CHUNK), jnp.float32),
            pltpu.VMEM((CCAP, CHUNK), jnp.float32),
            pltpu.VMEM((SCAP,), jnp.float32),
            pltpu.VMEM((SCAP,), jnp.int32),
            pltpu.SemaphoreType.DMA,
            pltpu.SemaphoreType.DMA,
            pltpu.SemaphoreType.DMA,
            pltpu.SemaphoreType.DMA,
        ],
    )
    def k3(cm_hbm, t_hbm, rows_hbm, selv_out, selp_out, nsel_out,
           cm_v0, cm_v1, t_v0, t_v1, ids1_v, ids2_v0, ids2_v1, nch_v,
           cand_v0, cand_v1, selv_v, selp_v, csem0, csem1, gsem, wsem):
        wid = lax.axis_index("s") * NC_SC + lax.axis_index("c")
        b0 = wid
        b1 = wid + NW
        cm_bufs = (cm_v0, cm_v1)
        t_bufs = (t_v0, t_v1)
        ids2_bufs = (ids2_v0, ids2_v1)
        cand_bufs = (cand_v0, cand_v1)
        cm0 = pltpu.async_copy(cm_hbm.at[b0], cm_v0, csem0)
        cm1 = pltpu.async_copy(cm_hbm.at[b1], cm_v1, csem1)

        def compact(r, b, cm_wait):
            cm_v = cm_bufs[r]
            ids2_v = ids2_bufs[r]
            cand_v = cand_bufs[r]
            pltpu.sync_copy(t_hbm.at[b], t_bufs[r])
            tval = t_bufs[r][...]
            gbase = b * NCH

            def zbody(j, carry):
                ids1_v[pl.ds(j * 16, 16)] = jnp.zeros((16,), jnp.int32)
                return carry

            lax.fori_loop(0, CCAP // 16, zbody, jnp.int32(0))
            cm_wait.wait()

            def cbody(j, off):
                v = cm_v[pl.ds(j * 16, 16)]
                mask = v >= tval
                ids = gbase + j * 16 + lax.iota(jnp.int32, 16)
                offc = jnp.minimum(off, CCAP - 16)
                plsc.store_compressed(ids1_v.at[pl.ds(offc, 16)], ids,
                                      mask=mask)
                return off + jnp.max(plsc.all_reduce_population_count(mask))

            off = lax.fori_loop(0, NCH // 16, cbody, jnp.int32(0))
            nch = jnp.minimum(off, CCAP)
            for i in range(3):
                for j in range(8):
                    ids2_v[i, pl.ds(j * 16, 16)] = (
                        ids1_v[pl.ds(i * 128 + j * 16, 16)])
            gs = [pltpu.async_copy(rows_hbm.at[ids2_v.at[i]],
                                   cand_v.at[pl.ds(i * 128, 128)], gsem)
                  for i in range(3)]
            return gs, nch

        def select(r, b, gs, nch):
            ids2_v = ids2_bufs[r]
            cand_v = cand_bufs[r]
            tval = t_bufs[r][...]
            nslots = nch * CHUNK
            gsub = b * NCH
            for g in gs:
                g.wait()

            def sbody2(j, off):
                base = j * 16
                row = base // CHUNK
                col = base % CHUNK
                v = cand_v[row, pl.ds(col, 16)]
                s = base + lax.iota(jnp.int32, 16)
                mask = (v >= tval) & (s < nslots)
                crow = lax.shift_right_logical(s, 6)
                gid = plsc.load_gather(
                    ids2_v, [lax.shift_right_logical(crow, 7), crow & 127])
                pfi = (gid - gsub) * CHUNK + (s & 63)
                offc = jnp.minimum(off, SCAP - 16)
                plsc.store_compressed(selv_v.at[pl.ds(offc, 16)], v,
                                      mask=mask)
                plsc.store_compressed(selp_v.at[pl.ds(offc, 16)], pfi,
                                      mask=mask)
                return off + jnp.max(plsc.all_reduce_population_count(mask))

            nit = (nslots + 15) // 16
            off = lax.fori_loop(0, nit, sbody2, jnp.int32(0))
            nch_v[...] = jnp.full((16,), jnp.minimum(off, SCAP), jnp.int32)
            pltpu.sync_copy(nch_v, nsel_out.at[b])
            pltpu.sync_copy(selv_v, selv_out.at[b])
            pltpu.sync_copy(selp_v, selp_out.at[b])

        g0, n0 = compact(0, b0, cm0)
        g1, n1 = compact(1, b1, cm1)
        select(0, b0, g0, n0)
        select(1, b1, g1, n1)

    return k3(cm2, t_rep, rows)


def _k6_body(selv_ref, selp_ref, selvc_ref, selpc_ref, nsel_ref, boxes_ref,
             ts_ref, sc_ref, lb_ref, bx_ref):
    v = selv_ref[...]
    p = selp_ref[...]
    nsel = nsel_ref[:, :, 0:1]
    slot = lax.broadcasted_iota(jnp.int32, (SB, 1, SCAP), 2)
    v = jnp.where(slot < nsel, v, NEG)
    p = jnp.where(slot < nsel, p, jnp.int32(2**31 - 1))
    vcol = selvc_ref[...]
    pcol = selpc_ref[...]
    slotc = lax.broadcasted_iota(jnp.int32, (SB, SCAP, 1), 1)
    vcol = jnp.where(slotc < nsel, vcol, NEG)
    pcol = jnp.where(slotc < nsel, pcol, jnp.int32(2**31 - 1))
    beats = (vcol > v) | ((vcol == v) & (pcol < p))
    beaten = (v > vcol) | ((v == vcol) & (p < pcol))
    rank_row = jnp.sum(beats.astype(jnp.int32), axis=1, keepdims=True)
    rank_col = jnp.sum(beaten.astype(jnp.int32), axis=2, keepdims=True)
    ohc = rank_col == lax.broadcasted_iota(jnp.int32, (SB, SCAP, K), 2)
    sc_ref[...] = jnp.sum(jnp.where(ohc, vcol, 0.0), axis=1, keepdims=True)
    pf_row = jnp.sum(jnp.where(ohc, pcol, 0), axis=1, keepdims=True)
    lb_ref[...] = jnp.bitwise_and(pf_row, 1023)
    ohr = rank_row == lax.broadcasted_iota(jnp.int32, (SB, K, SCAP), 1)
    pf_col = jnp.sum(jnp.where(ohr, p, 0), axis=2, keepdims=True)
    q_col = lax.shift_right_logical(pf_col, 10)
    oh2 = (lax.broadcasted_iota(jnp.int32, (SB, K, Q), 2) == q_col)
    bxs = lax.dot_general(oh2.astype(jnp.float32), boxes_ref[...],
                          (((2,), (1,)), ((0,), (0,))),
                          preferred_element_type=jnp.float32,
                          precision=lax.Precision.HIGHEST)
    cx = bxs[:, :, 0:1]
    cy = bxs[:, :, 1:2]
    w = bxs[:, :, 2:3]
    h = bxs[:, :, 3:4]
    ts = ts_ref[...]
    ih = ts[:, :, 0:1]
    iw = ts[:, :, 1:2]
    bx_ref[...] = jnp.concatenate(
        [(cx - 0.5 * w) * iw, (cy - 0.5 * h) * ih,
         (cx + 0.5 * w) * iw, (cy + 0.5 * h) * ih], axis=2)


def _order_and_extract(selv3, selp3, selvc3, selpc3, nsel3, pred_boxes, ts3):
    return pl.pallas_call(
        _k6_body,
        grid=(B // SB,),
        in_specs=[
            pl.BlockSpec((SB, 1, SCAP), lambda b: (b, 0, 0)),
            pl.BlockSpec((SB, 1, SCAP), lambda b: (b, 0, 0)),
            pl.BlockSpec((SB, SCAP, 1), lambda b: (b, 0, 0)),
            pl.BlockSpec((SB, SCAP, 1), lambda b: (b, 0, 0)),
            pl.BlockSpec((SB, 1, 16), lambda b: (b, 0, 0)),
            pl.BlockSpec((SB, Q, 4), lambda b: (b, 0, 0)),
            pl.BlockSpec((SB, 1, 2), lambda b: (b, 0, 0)),
        ],
        out_specs=[
            pl.BlockSpec((SB, 1, K), lambda b: (b, 0, 0)),
            pl.BlockSpec((SB, 1, K), lambda b: (b, 0, 0)),
            pl.BlockSpec((SB, K, 4), lambda b: (b, 0, 0)),
        ],
        out_shape=[
            jax.ShapeDtypeStruct((B, 1, K), jnp.float32),
            jax.ShapeDtypeStruct((B, 1, K), jnp.int32),
            jax.ShapeDtypeStruct((B, K, 4), jnp.float32),
        ],
    )(selv3, selp3, selvc3, selpc3, nsel3, pred_boxes, ts3)


def kernel(pred_logits, pred_boxes, target_sizes, positive_map):
    probp, cm = _score_map(pred_logits, positive_map)
    cm2 = cm.reshape(B, NCH)
    t_rep = _chunk_threshold(cm2)
    rows = probp.reshape(B * NCH, CHUNK)
    selv, selp, nsel_rep = _sc_gather_select(cm2, t_rep, rows)
    scores3, labels3, boxes = _order_and_extract(
        selv.reshape(B, 1, SCAP), selp.reshape(B, 1, SCAP),
        selv.reshape(B, SCAP, 1), selp.reshape(B, SCAP, 1),
        nsel_rep.reshape(B, 1, 16), pred_boxes,
        target_sizes.reshape(B, 1, 2))
    return scores3.reshape(B, K), labels3.reshape(B, K), boxes


# speedup vs baseline: 2.4666x; 1.0470x over previous
import functools

import jax
import jax.numpy as jnp
from jax import lax
from jax.experimental import pallas as pl
from jax.experimental.pallas import tpu as pltpu
from jax.experimental.pallas import tpu_sc as plsc

B, Q, T = 64, 900, 256
C = 900
K = 300
PADC = 1024
CHUNK = 64
NCH = Q * PADC // CHUNK
CCAP = 384
NCAND = CCAP * CHUNK
SCAP = 384
FINF_BITS = 0x7F800000
NC_SC, NS_SC = 2, 16
NW = NC_SC * NS_SC
NEG = -jnp.inf
SB = 8


def _k1_body(logits_ref, pmap_ref, probp_ref, cm_ref):
    x = logits_ref[0]
    s = jax.nn.sigmoid(x)
    p = lax.dot_general(s, pmap_ref[...], (((1,), (1,)), ((), ())),
                        preferred_element_type=jnp.float32)
    pad = jnp.full((Q, PADC - C), NEG, jnp.float32)
    probp_ref[0] = jnp.concatenate([p, pad], axis=1)
    cms = [jnp.max(p[:, 64 * i:64 * (i + 1)], axis=1, keepdims=True)
           for i in range(14)]
    cms.append(jnp.max(p[:, 896:900], axis=1, keepdims=True))
    cms.append(jnp.full((Q, 1), NEG, jnp.float32))
    cm_ref[0] = jnp.concatenate(cms, axis=1)


def _score_map(pred_logits, positive_map):
    return pl.pallas_call(
        _k1_body,
        grid=(B,),
        in_specs=[
            pl.BlockSpec((1, Q, T), lambda b: (b, 0, 0)),
            pl.BlockSpec((Q, T), lambda b: (0, 0)),
        ],
        out_specs=[
            pl.BlockSpec((1, Q, PADC), lambda b: (b, 0, 0)),
            pl.BlockSpec((1, Q, 16), lambda b: (b, 0, 0)),
        ],
        out_shape=[
            jax.ShapeDtypeStruct((B, Q, PADC), jnp.float32),
            jax.ShapeDtypeStruct((B, Q, 16), jnp.float32),
        ],
    )(pred_logits, positive_map)


def _bisect_core(v, k):
    def step(_, lohi):
        lo, hi = lohi
        mid = lo + (hi - lo) // 2
        t = lax.bitcast_convert_type(mid, jnp.float32)
        cnt = jnp.sum((v >= t).astype(jnp.int32), axis=1, keepdims=True)
        ge = cnt >= k
        return jnp.where(ge, mid, lo), jnp.where(ge, hi, mid)

    lo = jnp.zeros((v.shape[0], 1), jnp.int32)
    hi = jnp.full((v.shape[0], 1), FINF_BITS, jnp.int32)
    lo, hi = lax.fori_loop(0, 31, step, (lo, hi))
    return lax.bitcast_convert_type(lo, jnp.float32)


def _k2_body(cm_ref, t_ref):
    t = _bisect_core(cm_ref[...], K)
    t_ref[...] = jnp.broadcast_to(t, (B, 16))


def _chunk_threshold(cm2):
    return pl.pallas_call(
        _k2_body,
        out_shape=jax.ShapeDtypeStruct((B, 16), jnp.float32),
    )(cm2)


def _k4_body(cand_ref, nch_ref, t_ref):
    slot = lax.broadcasted_iota(jnp.int32, (1, NCAND), 1)
    nsl = nch_ref[:, 0:1] * CHUNK
    v = jnp.where(slot < nsl, cand_ref[...], NEG)
    t = _bisect_core(v, K)
    t_ref[...] = jnp.broadcast_to(t, (B, 16))


def _cand_threshold(cand2, nch_rep):
    return pl.pallas_call(
        _k4_body,
        out_shape=jax.ShapeDtypeStruct((B, 16), jnp.float32),
    )(cand2, nch_rep)


def _sc_gather_select(cm2, t_rep, rows):
    mesh = plsc.VectorSubcoreMesh(core_axis_name="c", subcore_axis_name="s")

    @functools.partial(
        pl.kernel,
        mesh=mesh,
        compiler_params=pltpu.CompilerParams(
            needs_layout_passes=False, use_tc_tiling_on_sc=False),
        out_type=[
            jax.ShapeDtypeStruct((B, SCAP), jnp.float32),
            jax.ShapeDtypeStruct((B, SCAP), jnp.int32),
            jax.ShapeDtypeStruct((B, 16), jnp.int32),
        ],
        scratch_types=[
            pltpu.VMEM((NCH,), jnp.float32),
            pltpu.VMEM((NCH,), jnp.float32),
            pltpu.VMEM((16,), jnp.float32),
            pltpu.VMEM((16,), jnp.float32),
            pltpu.VMEM((CCAP,), jnp.int32),
            pltpu.VMEM((3, 128), jnp.int32),
            pltpu.VMEM((3, 128), jnp.int32),
            pltpu.VMEM((16,), jnp.int32),
            pltpu.VMEM((CCAP, CHUNK), jnp.float32),
            pltpu.VMEM((CCAP, CHUNK), jnp.float32),
            pltpu.VMEM((SCAP,), jnp.float32),
            pltpu.VMEM((SCAP,), jnp.int32),
            pltpu.SemaphoreType.DMA,
            pltpu.SemaphoreType.DMA,
            pltpu.SemaphoreType.DMA,
            pltpu.SemaphoreType.DMA,
        ],
    )
    def k3(cm_hbm, t_hbm, rows_hbm, selv_out, selp_out, nsel_out,
           cm_v0, cm_v1, t_v0, t_v1, ids1_v, ids2_v0, ids2_v1, nch_v,
           cand_v0, cand_v1, selv_v, selp_v, csem0, csem1, gsem, wsem):
        wid = lax.axis_index("s") * NC_SC + lax.axis_index("c")
        b0 = wid
        b1 = wid + NW
        cm_bufs = (cm_v0, cm_v1)
        t_bufs = (t_v0, t_v1)
        ids2_bufs = (ids2_v0, ids2_v1)
        cand_bufs = (cand_v0, cand_v1)
        cm0 = pltpu.async_copy(cm_hbm.at[b0], cm_v0, csem0)
        cm1 = pltpu.async_copy(cm_hbm.at[b1], cm_v1, csem1)

        def compact(r, b, cm_wait):
            cm_v = cm_bufs[r]
            ids2_v = ids2_bufs[r]
            cand_v = cand_bufs[r]
            pltpu.sync_copy(t_hbm.at[b], t_bufs[r])
            tval = t_bufs[r][...]
            gbase = b * NCH

            def zbody(j, carry):
                ids1_v[pl.ds(j * 16, 16)] = jnp.zeros((16,), jnp.int32)
                return carry

            lax.fori_loop(0, CCAP // 16, zbody, jnp.int32(0))
            cm_wait.wait()

            def cbody(j, off):
                v = cm_v[pl.ds(j * 16, 16)]
                mask = v >= tval
                ids = gbase + j * 16 + lax.iota(jnp.int32, 16)
                offc = jnp.minimum(off, CCAP - 16)
                plsc.store_compressed(ids1_v.at[pl.ds(offc, 16)], ids,
                                      mask=mask)
                return off + jnp.max(plsc.all_reduce_population_count(mask))

            off = lax.fori_loop(0, NCH // 16, cbody, jnp.int32(0))
            nch = jnp.minimum(off, CCAP)
            for i in range(3):
                for j in range(8):
                    ids2_v[i, pl.ds(j * 16, 16)] = (
                        ids1_v[pl.ds(i * 128 + j * 16, 16)])
            gs = [pltpu.async_copy(rows_hbm.at[ids2_v.at[i]],
                                   cand_v.at[pl.ds(i * 128, 128)], gsem)
                  for i in range(3)]
            return gs, nch

        def select(r, b, gs, nch):
            ids2_v = ids2_bufs[r]
            cand_v = cand_bufs[r]
            tval = t_bufs[r][...]
            nslots = nch * CHUNK
            gsub = b * NCH
            for g in gs:
                g.wait()

            def sbody2(j, off):
                base = j * 16
                row = base // CHUNK
                col = base % CHUNK
                v = cand_v[row, pl.ds(col, 16)]
                s = base + lax.iota(jnp.int32, 16)
                mask = (v >= tval) & (s < nslots)
                crow = lax.shift_right_logical(s, 6)
                gid = plsc.load_gather(
                    ids2_v, [lax.shift_right_logical(crow, 7), crow & 127])
                pfi = (gid - gsub) * CHUNK + (s & 63)
                offc = jnp.minimum(off, SCAP - 16)
                plsc.store_compressed(selv_v.at[pl.ds(offc, 16)], v,
                                      mask=mask)
                plsc.store_compressed(selp_v.at[pl.ds(offc, 16)], pfi,
                                      mask=mask)
                return off + jnp.max(plsc.all_reduce_population_count(mask))

            nit = (nslots + 15) // 16
            off = lax.fori_loop(0, nit, sbody2, jnp.int32(0))
            nch_v[...] = jnp.full((16,), jnp.minimum(off, SCAP), jnp.int32)
            pltpu.sync_copy(nch_v, nsel_out.at[b])
            pltpu.sync_copy(selv_v, selv_out.at[b])
            pltpu.sync_copy(selp_v, selp_out.at[b])

        g0, n0 = compact(0, b0, cm0)
        g1, n1 = compact(1, b1, cm1)
        select(0, b0, g0, n0)
        select(1, b1, g1, n1)

    return k3(cm2, t_rep, rows)


def _k6_body(selv_ref, selp_ref, selvc_ref, selpc_ref, nsel_ref, boxes_ref,
             ts_ref, sc_ref, lb_ref, bx_ref):
    v = selv_ref[...]
    p = selp_ref[...]
    nsel = nsel_ref[:, :, 0:1]
    slot = lax.broadcasted_iota(jnp.int32, (SB, 1, SCAP), 2)
    v = jnp.where(slot < nsel, v, NEG)
    p = jnp.where(slot < nsel, p, jnp.int32(2**31 - 1))
    vcol = selvc_ref[...]
    pcol = selpc_ref[...]
    slotc = lax.broadcasted_iota(jnp.int32, (SB, SCAP, 1), 1)
    vcol = jnp.where(slotc < nsel, vcol, NEG)
    pcol = jnp.where(slotc < nsel, pcol, jnp.int32(2**31 - 1))
    beats = (vcol > v) | ((vcol == v) & (pcol < p))
    bi = beats.astype(jnp.int32)
    rank_row = jnp.sum(bi, axis=1, keepdims=True)
    rank_col = SCAP - 1 - jnp.sum(bi, axis=2, keepdims=True)
    ohc = rank_col == lax.broadcasted_iota(jnp.int32, (SB, SCAP, K), 2)
    sc_ref[...] = jnp.sum(jnp.where(ohc, vcol, 0.0), axis=1, keepdims=True)
    pf_row = jnp.sum(jnp.where(ohc, pcol, 0), axis=1, keepdims=True)
    lb_ref[...] = jnp.bitwise_and(pf_row, 1023)
    ohr = rank_row == lax.broadcasted_iota(jnp.int32, (SB, K, SCAP), 1)
    pf_col = jnp.sum(jnp.where(ohr, p, 0), axis=2, keepdims=True)
    q_col = lax.shift_right_logical(pf_col, 10)
    oh2 = (lax.broadcasted_iota(jnp.int32, (SB, K, Q), 2) == q_col)
    bxs = lax.dot_general(oh2.astype(jnp.float32), boxes_ref[...],
                          (((2,), (1,)), ((0,), (0,))),
                          preferred_element_type=jnp.float32,
                          precision=lax.Precision.HIGHEST)
    cx = bxs[:, :, 0:1]
    cy = bxs[:, :, 1:2]
    w = bxs[:, :, 2:3]
    h = bxs[:, :, 3:4]
    ts = ts_ref[...]
    ih = ts[:, :, 0:1]
    iw = ts[:, :, 1:2]
    bx_ref[...] = jnp.concatenate(
        [(cx - 0.5 * w) * iw, (cy - 0.5 * h) * ih,
         (cx + 0.5 * w) * iw, (cy + 0.5 * h) * ih], axis=2)


def _order_and_extract(selv3, selp3, selvc3, selpc3, nsel3, pred_boxes, ts3):
    return pl.pallas_call(
        _k6_body,
        grid=(B // SB,),
        in_specs=[
            pl.BlockSpec((SB, 1, SCAP), lambda b: (b, 0, 0)),
            pl.BlockSpec((SB, 1, SCAP), lambda b: (b, 0, 0)),
            pl.BlockSpec((SB, SCAP, 1), lambda b: (b, 0, 0)),
            pl.BlockSpec((SB, SCAP, 1), lambda b: (b, 0, 0)),
            pl.BlockSpec((SB, 1, 16), lambda b: (b, 0, 0)),
            pl.BlockSpec((SB, Q, 4), lambda b: (b, 0, 0)),
            pl.BlockSpec((SB, 1, 2), lambda b: (b, 0, 0)),
        ],
        out_specs=[
            pl.BlockSpec((SB, 1, K), lambda b: (b, 0, 0)),
            pl.BlockSpec((SB, 1, K), lambda b: (b, 0, 0)),
            pl.BlockSpec((SB, K, 4), lambda b: (b, 0, 0)),
        ],
        out_shape=[
            jax.ShapeDtypeStruct((B, 1, K), jnp.float32),
            jax.ShapeDtypeStruct((B, 1, K), jnp.int32),
            jax.ShapeDtypeStruct((B, K, 4), jnp.float32),
        ],
    )(selv3, selp3, selvc3, selpc3, nsel3, pred_boxes, ts3)


def kernel(pred_logits, pred_boxes, target_sizes, positive_map):
    probp, cm = _score_map(pred_logits, positive_map)
    cm2 = cm.reshape(B, NCH)
    t_rep = _chunk_threshold(cm2)
    rows = probp.reshape(B * NCH, CHUNK)
    selv, selp, nsel_rep = _sc_gather_select(cm2, t_rep, rows)
    scores3, labels3, boxes = _order_and_extract(
        selv.reshape(B, 1, SCAP), selp.reshape(B, 1, SCAP),
        selv.reshape(B, SCAP, 1), selp.reshape(B, SCAP, 1),
        nsel_rep.reshape(B, 1, 16), pred_boxes,
        target_sizes.reshape(B, 1, 2))
    return scores3.reshape(B, K), labels3.reshape(B, K), boxes


# speedup vs baseline: 2.4697x; 1.0012x over previous
import functools

import jax
import jax.numpy as jnp
from jax import lax
from jax.experimental import pallas as pl
from jax.experimental.pallas import tpu as pltpu
from jax.experimental.pallas import tpu_sc as plsc

B, Q, T = 64, 900, 256
C = 900
K = 300
PADC = 1024
CHUNK = 64
NCH = Q * PADC // CHUNK
CCAP = 384
NCAND = CCAP * CHUNK
SCAP = 384
FINF_BITS = 0x7F800000
NC_SC, NS_SC = 2, 16
NW = NC_SC * NS_SC
NEG = -jnp.inf
SB = 16


def _k1_body(logits_ref, pmap_ref, probp_ref, cm_ref):
    x = logits_ref[0]
    s = jax.nn.sigmoid(x)
    p = lax.dot_general(s, pmap_ref[...], (((1,), (1,)), ((), ())),
                        preferred_element_type=jnp.float32)
    pad = jnp.full((Q, PADC - C), NEG, jnp.float32)
    probp_ref[0] = jnp.concatenate([p, pad], axis=1)
    cms = [jnp.max(p[:, 64 * i:64 * (i + 1)], axis=1, keepdims=True)
           for i in range(14)]
    cms.append(jnp.max(p[:, 896:900], axis=1, keepdims=True))
    cms.append(jnp.full((Q, 1), NEG, jnp.float32))
    cm_ref[0] = jnp.concatenate(cms, axis=1)


def _score_map(pred_logits, positive_map):
    return pl.pallas_call(
        _k1_body,
        grid=(B,),
        in_specs=[
            pl.BlockSpec((1, Q, T), lambda b: (b, 0, 0)),
            pl.BlockSpec((Q, T), lambda b: (0, 0)),
        ],
        out_specs=[
            pl.BlockSpec((1, Q, PADC), lambda b: (b, 0, 0)),
            pl.BlockSpec((1, Q, 16), lambda b: (b, 0, 0)),
        ],
        out_shape=[
            jax.ShapeDtypeStruct((B, Q, PADC), jnp.float32),
            jax.ShapeDtypeStruct((B, Q, 16), jnp.float32),
        ],
    )(pred_logits, positive_map)


def _bisect_core(v, k):
    def step(_, lohi):
        lo, hi = lohi
        mid = lo + (hi - lo) // 2
        t = lax.bitcast_convert_type(mid, jnp.float32)
        cnt = jnp.sum((v >= t).astype(jnp.int32), axis=1, keepdims=True)
        ge = cnt >= k
        return jnp.where(ge, mid, lo), jnp.where(ge, hi, mid)

    lo = jnp.zeros((v.shape[0], 1), jnp.int32)
    hi = jnp.full((v.shape[0], 1), FINF_BITS, jnp.int32)
    lo, hi = lax.fori_loop(0, 31, step, (lo, hi))
    return lax.bitcast_convert_type(lo, jnp.float32)


def _k2_body(cm_ref, t_ref):
    t = _bisect_core(cm_ref[...], K)
    t_ref[...] = jnp.broadcast_to(t, (B, 16))


def _chunk_threshold(cm2):
    return pl.pallas_call(
        _k2_body,
        out_shape=jax.ShapeDtypeStruct((B, 16), jnp.float32),
    )(cm2)


def _k4_body(cand_ref, nch_ref, t_ref):
    slot = lax.broadcasted_iota(jnp.int32, (1, NCAND), 1)
    nsl = nch_ref[:, 0:1] * CHUNK
    v = jnp.where(slot < nsl, cand_ref[...], NEG)
    t = _bisect_core(v, K)
    t_ref[...] = jnp.broadcast_to(t, (B, 16))


def _cand_threshold(cand2, nch_rep):
    return pl.pallas_call(
        _k4_body,
        out_shape=jax.ShapeDtypeStruct((B, 16), jnp.float32),
    )(cand2, nch_rep)


def _sc_gather_select(cm2, t_rep, rows):
    mesh = plsc.VectorSubcoreMesh(core_axis_name="c", subcore_axis_name="s")

    @functools.partial(
        pl.kernel,
        mesh=mesh,
        compiler_params=pltpu.CompilerParams(
            needs_layout_passes=False, use_tc_tiling_on_sc=False),
        out_type=[
            jax.ShapeDtypeStruct((B, SCAP), jnp.float32),
            jax.ShapeDtypeStruct((B, SCAP), jnp.int32),
            jax.ShapeDtypeStruct((B, 16), jnp.int32),
        ],
        scratch_types=[
            pltpu.VMEM((NCH,), jnp.float32),
            pltpu.VMEM((NCH,), jnp.float32),
            pltpu.VMEM((16,), jnp.float32),
            pltpu.VMEM((16,), jnp.float32),
            pltpu.VMEM((CCAP,), jnp.int32),
            pltpu.VMEM((3, 128), jnp.int32),
            pltpu.VMEM((3, 128), jnp.int32),
            pltpu.VMEM((16,), jnp.int32),
            pltpu.VMEM((CCAP, CHUNK), jnp.float32),
            pltpu.VMEM((CCAP, CHUNK), jnp.float32),
            pltpu.VMEM((SCAP,), jnp.float32),
            pltpu.VMEM((SCAP,), jnp.int32),
            pltpu.SemaphoreType.DMA,
            pltpu.SemaphoreType.DMA,
            pltpu.SemaphoreType.DMA,
            pltpu.SemaphoreType.DMA,
        ],
    )
    def k3(cm_hbm, t_hbm, rows_hbm, selv_out, selp_out, nsel_out,
           cm_v0, cm_v1, t_v0, t_v1, ids1_v, ids2_v0, ids2_v1, nch_v,
           cand_v0, cand_v1, selv_v, selp_v, csem0, csem1, gsem, wsem):
        wid = lax.axis_index("s") * NC_SC + lax.axis_index("c")
        b0 = wid
        b1 = wid + NW
        cm_bufs = (cm_v0, cm_v1)
        t_bufs = (t_v0, t_v1)
        ids2_bufs = (ids2_v0, ids2_v1)
        cand_bufs = (cand_v0, cand_v1)
        cm0 = pltpu.async_copy(cm_hbm.at[b0], cm_v0, csem0)
        cm1 = pltpu.async_copy(cm_hbm.at[b1], cm_v1, csem1)

        def compact(r, b, cm_wait):
            cm_v = cm_bufs[r]
            ids2_v = ids2_bufs[r]
            cand_v = cand_bufs[r]
            pltpu.sync_copy(t_hbm.at[b], t_bufs[r])
            tval = t_bufs[r][...]
            gbase = b * NCH

            def zbody(j, carry):
                ids1_v[pl.ds(j * 16, 16)] = jnp.zeros((16,), jnp.int32)
                return carry

            lax.fori_loop(0, CCAP // 16, zbody, jnp.int32(0))
            cm_wait.wait()

            def cbody(j, off):
                v = cm_v[pl.ds(j * 16, 16)]
                mask = v >= tval
                ids = gbase + j * 16 + lax.iota(jnp.int32, 16)
                offc = jnp.minimum(off, CCAP - 16)
                plsc.store_compressed(ids1_v.at[pl.ds(offc, 16)], ids,
                                      mask=mask)
                return off + jnp.max(plsc.all_reduce_population_count(mask))

            off = lax.fori_loop(0, NCH // 16, cbody, jnp.int32(0))
            nch = jnp.minimum(off, CCAP)
            for i in range(3):
                for j in range(8):
                    ids2_v[i, pl.ds(j * 16, 16)] = (
                        ids1_v[pl.ds(i * 128 + j * 16, 16)])
            gs = [pltpu.async_copy(rows_hbm.at[ids2_v.at[i]],
                                   cand_v.at[pl.ds(i * 128, 128)], gsem)
                  for i in range(3)]
            return gs, nch

        def select(r, b, gs, nch):
            ids2_v = ids2_bufs[r]
            cand_v = cand_bufs[r]
            tval = t_bufs[r][...]
            nslots = nch * CHUNK
            gsub = b * NCH
            for g in gs:
                g.wait()

            def sbody2(j, off):
                base = j * 16
                row = base // CHUNK
                col = base % CHUNK
                v = cand_v[row, pl.ds(col, 16)]
                s = base + lax.iota(jnp.int32, 16)
                mask = (v >= tval) & (s < nslots)
                crow = lax.shift_right_logical(s, 6)
                gid = plsc.load_gather(
                    ids2_v, [lax.shift_right_logical(crow, 7), crow & 127])
                pfi = (gid - gsub) * CHUNK + (s & 63)
                offc = jnp.minimum(off, SCAP - 16)
                plsc.store_compressed(selv_v.at[pl.ds(offc, 16)], v,
                                      mask=mask)
                plsc.store_compressed(selp_v.at[pl.ds(offc, 16)], pfi,
                                      mask=mask)
                return off + jnp.max(plsc.all_reduce_population_count(mask))

            nit = (nslots + 15) // 16
            off = lax.fori_loop(0, nit, sbody2, jnp.int32(0))
            nch_v[...] = jnp.full((16,), jnp.minimum(off, SCAP), jnp.int32)
            pltpu.sync_copy(nch_v, nsel_out.at[b])
            pltpu.sync_copy(selv_v, selv_out.at[b])
            pltpu.sync_copy(selp_v, selp_out.at[b])

        g0, n0 = compact(0, b0, cm0)
        g1, n1 = compact(1, b1, cm1)
        select(0, b0, g0, n0)
        select(1, b1, g1, n1)

    return k3(cm2, t_rep, rows)


def _k6_body(selv_ref, selp_ref, selvc_ref, selpc_ref, nsel_ref, boxes_ref,
             ts_ref, sc_ref, lb_ref, bx_ref):
    v = selv_ref[...]
    p = selp_ref[...]
    nsel = nsel_ref[:, :, 0:1]
    slot = lax.broadcasted_iota(jnp.int32, (SB, 1, SCAP), 2)
    v = jnp.where(slot < nsel, v, NEG)
    p = jnp.where(slot < nsel, p, jnp.int32(2**31 - 1))
    vcol = selvc_ref[...]
    pcol = selpc_ref[...]
    slotc = lax.broadcasted_iota(jnp.int32, (SB, SCAP, 1), 1)
    vcol = jnp.where(slotc < nsel, vcol, NEG)
    pcol = jnp.where(slotc < nsel, pcol, jnp.int32(2**31 - 1))
    beats = (vcol > v) | ((vcol == v) & (pcol < p))
    bi = beats.astype(jnp.int32)
    rank_row = jnp.sum(bi, axis=1, keepdims=True)
    rank_col = SCAP - 1 - jnp.sum(bi, axis=2, keepdims=True)
    ohc = rank_col == lax.broadcasted_iota(jnp.int32, (SB, SCAP, K), 2)
    sc_ref[...] = jnp.sum(jnp.where(ohc, vcol, 0.0), axis=1, keepdims=True)
    pf_row = jnp.sum(jnp.where(ohc, pcol, 0), axis=1, keepdims=True)
    lb_ref[...] = jnp.bitwise_and(pf_row, 1023)
    ohr = rank_row == lax.broadcasted_iota(jnp.int32, (SB, K, SCAP), 1)
    pf_col = jnp.sum(jnp.where(ohr, p, 0), axis=2, keepdims=True)
    q_col = lax.shift_right_logical(pf_col, 10)
    oh2 = (lax.broadcasted_iota(jnp.int32, (SB, K, Q), 2) == q_col)
    bxs = lax.dot_general(oh2.astype(jnp.float32), boxes_ref[...],
                          (((2,), (1,)), ((0,), (0,))),
                          preferred_element_type=jnp.float32,
                          precision=lax.Precision.HIGHEST)
    cx = bxs[:, :, 0:1]
    cy = bxs[:, :, 1:2]
    w = bxs[:, :, 2:3]
    h = bxs[:, :, 3:4]
    ts = ts_ref[...]
    ih = ts[:, :, 0:1]
    iw = ts[:, :, 1:2]
    bx_ref[...] = jnp.concatenate(
        [(cx - 0.5 * w) * iw, (cy - 0.5 * h) * ih,
         (cx + 0.5 * w) * iw, (cy + 0.5 * h) * ih], axis=2)


def _order_and_extract(selv3, selp3, selvc3, selpc3, nsel3, pred_boxes, ts3):
    return pl.pallas_call(
        _k6_body,
        grid=(B // SB,),
        in_specs=[
            pl.BlockSpec((SB, 1, SCAP), lambda b: (b, 0, 0)),
            pl.BlockSpec((SB, 1, SCAP), lambda b: (b, 0, 0)),
            pl.BlockSpec((SB, SCAP, 1), lambda b: (b, 0, 0)),
            pl.BlockSpec((SB, SCAP, 1), lambda b: (b, 0, 0)),
            pl.BlockSpec((SB, 1, 16), lambda b: (b, 0, 0)),
            pl.BlockSpec((SB, Q, 4), lambda b: (b, 0, 0)),
            pl.BlockSpec((SB, 1, 2), lambda b: (b, 0, 0)),
        ],
        out_specs=[
            pl.BlockSpec((SB, 1, K), lambda b: (b, 0, 0)),
            pl.BlockSpec((SB, 1, K), lambda b: (b, 0, 0)),
            pl.BlockSpec((SB, K, 4), lambda b: (b, 0, 0)),
        ],
        out_shape=[
            jax.ShapeDtypeStruct((B, 1, K), jnp.float32),
            jax.ShapeDtypeStruct((B, 1, K), jnp.int32),
            jax.ShapeDtypeStruct((B, K, 4), jnp.float32),
        ],
    )(selv3, selp3, selvc3, selpc3, nsel3, pred_boxes, ts3)


def kernel(pred_logits, pred_boxes, target_sizes, positive_map):
    probp, cm = _score_map(pred_logits, positive_map)
    cm2 = cm.reshape(B, NCH)
    t_rep = _chunk_threshold(cm2)
    rows = probp.reshape(B * NCH, CHUNK)
    selv, selp, nsel_rep = _sc_gather_select(cm2, t_rep, rows)
    scores3, labels3, boxes = _order_and_extract(
        selv.reshape(B, 1, SCAP), selp.reshape(B, 1, SCAP),
        selv.reshape(B, SCAP, 1), selp.reshape(B, SCAP, 1),
        nsel_rep.reshape(B, 1, 16), pred_boxes,
        target_sizes.reshape(B, 1, 2))
    return scores3.reshape(B, K), labels3.reshape(B, K), boxes
